# baseline (speedup 1.0000x reference)
import jax
import jax.numpy as jnp
from jax.experimental import pallas as pl
from jax.experimental.pallas import tpu as pltpu


def _take_first(x_hbm, o_ref, buf, sem):
    copy = pltpu.make_async_copy(x_hbm.at[pl.ds(0, 128)], buf, sem)
    copy.start()
    copy.wait()
    o_ref[0] = buf[0]


def kernel(x):
    out = pl.pallas_call(
        _take_first,
        out_shape=jax.ShapeDtypeStruct((1,), jnp.float32),
        in_specs=[pl.BlockSpec(memory_space=pl.MemorySpace.ANY)],
        out_specs=pl.BlockSpec(memory_space=pltpu.SMEM),
        scratch_shapes=[pltpu.SMEM((128,), jnp.float32), pltpu.SemaphoreType.DMA],
    )(x)
    return out.reshape(())


# speedup vs baseline: 1.1019x; 1.1019x over previous
import jax
import jax.numpy as jnp
from jax.experimental import pallas as pl
from jax.experimental.pallas import tpu as pltpu


def _take_first(x_ref, o_ref):
    o_ref[...] = x_ref[pl.ds(0, 1)]


def kernel(x):
    out = pl.pallas_call(
        _take_first,
        out_shape=jax.ShapeDtypeStruct((1,), jnp.float32),
        grid=(1,),
        in_specs=[pl.BlockSpec((128,), lambda i: (0,))],
        out_specs=pl.BlockSpec((1,), lambda i: (0,)),
    )(x)
    return out.reshape(())


# speedup vs baseline: 1.1365x; 1.0314x over previous
import jax
import jax.numpy as jnp
from jax.experimental import pallas as pl


def _take_first(x_ref, o_ref):
    o_ref[...] = x_ref[pl.ds(0, 1)]


def kernel(x):
    out = pl.pallas_call(
        _take_first,
        out_shape=jax.ShapeDtypeStruct((1,), jnp.float32),
        grid=(1,),
        in_specs=[pl.BlockSpec((128,), lambda i: (0,))],
        out_specs=pl.BlockSpec((1,), lambda i: (0,)),
    )(x)
    return out.reshape(())
